# baseline (speedup 1.0000x reference)
import functools

import jax
import jax.numpy as jnp
from jax import lax
from jax.experimental import pallas as pl
from jax.experimental.pallas import tpu as pltpu
from jax.experimental.pallas import tpu_sc as plsc

N_NODES = 10000
N_PAD = 10240
N_EDGES = 320000
D_FEAT = 128
D_EDGE = 16
D_MSG = 2 * D_FEAT


_XB = 2000


def _mm2_body(x_ref, w1_ref, w2_ref, o1_ref, o2_ref):
    x = x_ref[...]
    o1_ref[...] = jnp.dot(x, w1_ref[...], preferred_element_type=jnp.float32)
    o2_ref[...] = jnp.dot(x, w2_ref[...], preferred_element_type=jnp.float32)


def _mm2(x, w1, w2):
    grid = (N_NODES // _XB,)
    return pl.pallas_call(
        _mm2_body,
        grid=grid,
        in_specs=[
            pl.BlockSpec((_XB, D_FEAT), lambda i: (i, 0)),
            pl.BlockSpec((D_FEAT, D_MSG), lambda i: (0, 0)),
            pl.BlockSpec((D_FEAT, D_MSG), lambda i: (0, 0)),
        ],
        out_specs=[
            pl.BlockSpec((_XB, D_MSG), lambda i: (i, 0)),
            pl.BlockSpec((_XB, D_MSG), lambda i: (i, 0)),
        ],
        out_shape=[
            jax.ShapeDtypeStruct((N_NODES, D_MSG), jnp.float32),
            jax.ShapeDtypeStruct((N_NODES, D_MSG), jnp.float32),
        ],
    )(x, w1, w2)


_EB = 3200


def _eproj_body(ea_ref, w3_ref, b_ref, o_ref):
    o_ref[...] = (
        jnp.dot(ea_ref[...], w3_ref[...], preferred_element_type=jnp.float32)
        + b_ref[...]
    )


def _eproj(ea, w3, b):
    grid = (N_EDGES // _EB,)
    return pl.pallas_call(
        _eproj_body,
        grid=grid,
        in_specs=[
            pl.BlockSpec((_EB, D_EDGE), lambda i: (i, 0)),
            pl.BlockSpec((D_EDGE, D_MSG), lambda i: (0, 0)),
            pl.BlockSpec((1, D_MSG), lambda i: (0, 0)),
        ],
        out_specs=pl.BlockSpec((_EB, D_MSG), lambda i: (i, 0)),
        out_shape=jax.ShapeDtypeStruct((N_EDGES, D_MSG), jnp.float32),
    )(ea, w3, b)


_B = 16
_CB = 26
_CHUNK = _B * _CB
_NCHUNK = 24


def _sc_edge_kernel(xw1, xw2, eproj, src, dst, zeros):
    info = plsc.get_sparse_core_info()
    nc, ns = info.num_cores, info.num_subcores
    nw = nc * ns
    epw = N_EDGES // nw
    rows_per_sub = N_PAD // ns

    mesh = plsc.VectorSubcoreMesh(core_axis_name="c", subcore_axis_name="s")

    @functools.partial(
        pl.kernel,
        mesh=mesh,
        out_type=jax.ShapeDtypeStruct((nc, N_PAD, D_FEAT), jnp.float32),
        scratch_types=[
            pltpu.VMEM((_CHUNK,), jnp.int32),
            pltpu.VMEM((_CHUNK,), jnp.int32),
            pltpu.VMEM((2, _B, D_MSG), jnp.float32),
            pltpu.VMEM((2, _B, D_MSG), jnp.float32),
            pltpu.VMEM((2, _B, D_MSG), jnp.float32),
            pltpu.VMEM((_B, D_FEAT), jnp.float32),
            pltpu.SemaphoreType.DMA,
            pltpu.SemaphoreType.DMA,
            pltpu.VMEM_SHARED((N_PAD, D_FEAT), jnp.float32),
        ],
    )
    def k(xw1_hbm, xw2_hbm, ep_hbm, src_hbm, dst_hbm, zero_hbm, out_hbm,
          srci_v, dsti_v, r1_v, r2_v, ep_v, m_v, sem0, sem1, agg_sh):
        c = lax.axis_index("c")
        s = lax.axis_index("s")
        wid = c * ns + s
        base_w = wid * epw
        sems = (sem0, sem1)

        stripe = pl.ds(s * rows_per_sub, rows_per_sub)
        pltpu.sync_copy(zero_hbm.at[stripe], agg_sh.at[stripe])
        plsc.subcore_barrier()

        def issue(cbase, i, slot):
            svec = srci_v[pl.ds(i * _B, _B)]
            dvec = dsti_v[pl.ds(i * _B, _B)]
            sem = sems[slot]
            pltpu.async_copy(xw1_hbm.at[svec], r1_v.at[slot], sem)
            pltpu.async_copy(xw2_hbm.at[dvec], r2_v.at[slot], sem)
            pltpu.async_copy(ep_hbm.at[pl.ds(cbase + i * _B, _B)],
                             ep_v.at[slot], sem)

        def drain(slot):
            sem = sems[slot]
            pltpu.make_async_copy(xw1_hbm.at[pl.ds(0, _B)], r1_v.at[slot],
                                  sem).wait()
            pltpu.make_async_copy(xw2_hbm.at[pl.ds(0, _B)], r2_v.at[slot],
                                  sem).wait()
            pltpu.make_async_copy(ep_hbm.at[pl.ds(0, _B)], ep_v.at[slot],
                                  sem).wait()

        def compute_scatter(i, slot):
            @plsc.parallel_loop(0, _B, unroll=4)
            def _(e):
                for j in range(D_FEAT // 16):
                    gsl = pl.ds(j * 16, 16)
                    csl = pl.ds(D_FEAT + j * 16, 16)
                    g = r1_v[slot, e, gsl] + r2_v[slot, e, gsl] + ep_v[slot, e, gsl]
                    cc = r1_v[slot, e, csl] + r2_v[slot, e, csl] + ep_v[slot, e, csl]
                    denom = (1.0 + jnp.exp(-g)) * (1.0 + jnp.exp(-cc))
                    m_v[e, gsl] = cc / denom
            dvec = dsti_v[pl.ds(i * _B, _B)]
            pltpu.sync_copy(m_v, agg_sh.at[dvec], add=True)

        def chunk_body(ci, carry):
            cbase = base_w + ci * _CHUNK
            pltpu.sync_copy(src_hbm.at[pl.ds(cbase, _CHUNK)], srci_v)
            pltpu.sync_copy(dst_hbm.at[pl.ds(cbase, _CHUNK)], dsti_v)
            issue(cbase, 0, 0)

            def pair_body(p, carry2):
                drain(0)
                issue(cbase, 2 * p + 1, 1)
                compute_scatter(2 * p, 0)
                drain(1)

                @pl.when(p < _CB // 2 - 1)
                def _():
                    issue(cbase, 2 * p + 2, 0)

                compute_scatter(2 * p + 1, 1)
                return carry2

            lax.fori_loop(0, _CB // 2, pair_body, 0)
            return carry

        lax.fori_loop(0, _NCHUNK, chunk_body, 0)

        tbase = base_w + _NCHUNK * _CHUNK
        pltpu.sync_copy(src_hbm.at[pl.ds(tbase, _B)], srci_v.at[pl.ds(0, _B)])
        pltpu.sync_copy(dst_hbm.at[pl.ds(tbase, _B)], dsti_v.at[pl.ds(0, _B)])
        issue(tbase, 0, 0)
        drain(0)
        compute_scatter(0, 0)

        plsc.subcore_barrier()
        pltpu.sync_copy(agg_sh.at[stripe], out_hbm.at[c, stripe])

    return k(xw1, xw2, eproj, src, dst, zeros)


def _post_body(x_ref, a0_ref, a1_ref, w_ref, b_ref, o_ref):
    agg = a0_ref[...] + a1_ref[...]
    o_ref[...] = (
        x_ref[...]
        + jnp.dot(agg, w_ref[...], preferred_element_type=jnp.float32)
        + b_ref[...]
    )


def _post(x, a0, a1, w_out, b_out):
    grid = (N_NODES // _XB,)
    return pl.pallas_call(
        _post_body,
        grid=grid,
        in_specs=[
            pl.BlockSpec((_XB, D_FEAT), lambda i: (i, 0)),
            pl.BlockSpec((_XB, D_FEAT), lambda i: (i, 0)),
            pl.BlockSpec((_XB, D_FEAT), lambda i: (i, 0)),
            pl.BlockSpec((D_FEAT, D_FEAT), lambda i: (0, 0)),
            pl.BlockSpec((1, D_FEAT), lambda i: (0, 0)),
        ],
        out_specs=pl.BlockSpec((_XB, D_FEAT), lambda i: (i, 0)),
        out_shape=jax.ShapeDtypeStruct((N_NODES, D_FEAT), jnp.float32),
    )(x, a0, a1, w_out, b_out)


def kernel(x, edge_index, edge_attr, W_msg, b_msg, W_out, b_out):
    x = x.astype(jnp.float32)
    src = edge_index[0].astype(jnp.int32)
    dst = edge_index[1].astype(jnp.int32)

    w1 = W_msg[:D_FEAT]
    w2 = W_msg[D_FEAT : 2 * D_FEAT]
    w3 = W_msg[2 * D_FEAT :]

    xw1, xw2 = _mm2(x, w1, w2)
    ep = _eproj(edge_attr, w3, b_msg.reshape(1, D_MSG))

    zeros = jnp.zeros((N_PAD, D_FEAT), jnp.float32)
    aggs = _sc_edge_kernel(xw1, xw2, ep, src, dst, zeros)

    return _post(
        x, aggs[0, :N_NODES], aggs[1, :N_NODES], W_out, b_out.reshape(1, D_FEAT)
    )


# speedup vs baseline: 1.2253x; 1.2253x over previous
import functools

import jax
import jax.numpy as jnp
from jax import lax
from jax.experimental import pallas as pl
from jax.experimental.pallas import tpu as pltpu
from jax.experimental.pallas import tpu_sc as plsc

N_NODES = 10000
N_PAD = 10240
N_EDGES = 320000
D_FEAT = 128
D_EDGE = 16
D_MSG = 2 * D_FEAT


_XB = 2000


def _mm2_body(x_ref, w1_ref, w2_ref, o1_ref, o2_ref):
    x = x_ref[...]
    o1_ref[...] = jnp.dot(x, w1_ref[...], preferred_element_type=jnp.float32)
    o2_ref[...] = jnp.dot(x, w2_ref[...], preferred_element_type=jnp.float32)


def _mm2(x, w1, w2):
    grid = (N_NODES // _XB,)
    return pl.pallas_call(
        _mm2_body,
        grid=grid,
        in_specs=[
            pl.BlockSpec((_XB, D_FEAT), lambda i: (i, 0)),
            pl.BlockSpec((D_FEAT, D_MSG), lambda i: (0, 0)),
            pl.BlockSpec((D_FEAT, D_MSG), lambda i: (0, 0)),
        ],
        out_specs=[
            pl.BlockSpec((_XB, D_MSG), lambda i: (i, 0)),
            pl.BlockSpec((_XB, D_MSG), lambda i: (i, 0)),
        ],
        out_shape=[
            jax.ShapeDtypeStruct((N_NODES, D_MSG), jnp.float32),
            jax.ShapeDtypeStruct((N_NODES, D_MSG), jnp.float32),
        ],
    )(x, w1, w2)


_EB = 3200


def _eproj_body(ea_ref, w3_ref, b_ref, o_ref):
    o_ref[...] = (
        jnp.dot(ea_ref[...], w3_ref[...], preferred_element_type=jnp.float32)
        + b_ref[...]
    )


def _eproj(ea, w3, b):
    grid = (N_EDGES // _EB,)
    return pl.pallas_call(
        _eproj_body,
        grid=grid,
        in_specs=[
            pl.BlockSpec((_EB, D_EDGE), lambda i: (i, 0)),
            pl.BlockSpec((D_EDGE, D_MSG), lambda i: (0, 0)),
            pl.BlockSpec((1, D_MSG), lambda i: (0, 0)),
        ],
        out_specs=pl.BlockSpec((_EB, D_MSG), lambda i: (i, 0)),
        out_shape=jax.ShapeDtypeStruct((N_EDGES, D_MSG), jnp.float32),
    )(ea, w3, b)


_B = 16
_CB = 26
_CHUNK = _B * _CB
_NCHUNK = 24


def _sc_edge_kernel(xw1, xw2, eproj, src, dst, zeros):
    info = plsc.get_sparse_core_info()
    nc, ns = info.num_cores, info.num_subcores
    nw = nc * ns
    epw = N_EDGES // nw
    rows_per_sub = N_PAD // ns

    mesh = plsc.VectorSubcoreMesh(core_axis_name="c", subcore_axis_name="s")

    @functools.partial(
        pl.kernel,
        mesh=mesh,
        out_type=jax.ShapeDtypeStruct((nc, N_PAD, D_FEAT), jnp.float32),
        scratch_types=[
            pltpu.VMEM((_CHUNK,), jnp.int32),
            pltpu.VMEM((_CHUNK,), jnp.int32),
            pltpu.VMEM((2, _B, D_MSG), jnp.float32),
            pltpu.VMEM((2, _B, D_MSG), jnp.float32),
            pltpu.VMEM((2, _B, D_MSG), jnp.float32),
            pltpu.VMEM((2, _B, D_FEAT), jnp.float32),
            pltpu.SemaphoreType.DMA,
            pltpu.SemaphoreType.DMA,
            pltpu.SemaphoreType.DMA,
            pltpu.SemaphoreType.DMA,
            pltpu.VMEM_SHARED((N_PAD, D_FEAT), jnp.float32),
        ],
    )
    def k(xw1_hbm, xw2_hbm, ep_hbm, src_hbm, dst_hbm, zero_hbm, out_hbm,
          srci_v, dsti_v, r1_v, r2_v, ep_v, m_v, sem0, sem1, ssem0, ssem1,
          agg_sh):
        c = lax.axis_index("c")
        s = lax.axis_index("s")
        wid = c * ns + s
        base_w = wid * epw
        sems = (sem0, sem1)
        ssems = (ssem0, ssem1)

        stripe = pl.ds(s * rows_per_sub, rows_per_sub)
        pltpu.sync_copy(zero_hbm.at[stripe], agg_sh.at[stripe])
        plsc.subcore_barrier()

        def issue(cbase, i, slot):
            svec = srci_v[pl.ds(i * _B, _B)]
            dvec = dsti_v[pl.ds(i * _B, _B)]
            sem = sems[slot]
            pltpu.async_copy(xw1_hbm.at[svec], r1_v.at[slot], sem)
            pltpu.async_copy(xw2_hbm.at[dvec], r2_v.at[slot], sem)
            pltpu.async_copy(ep_hbm.at[pl.ds(cbase + i * _B, _B)],
                             ep_v.at[slot], sem)

        def drain(slot):
            sem = sems[slot]
            pltpu.make_async_copy(xw1_hbm.at[pl.ds(0, _B)], r1_v.at[slot],
                                  sem).wait()
            pltpu.make_async_copy(xw2_hbm.at[pl.ds(0, _B)], r2_v.at[slot],
                                  sem).wait()
            pltpu.make_async_copy(ep_hbm.at[pl.ds(0, _B)], ep_v.at[slot],
                                  sem).wait()

        def wait_scatter(slot):
            pltpu.make_async_copy(zero_hbm.at[pl.ds(0, _B)], m_v.at[slot],
                                  ssems[slot]).wait()

        def compute_scatter(i, slot, wait_ok):
            @pl.when(wait_ok)
            def _():
                wait_scatter(slot)

            @plsc.parallel_loop(0, _B, unroll=2)
            def _(e):
                for j in range(D_FEAT // 16):
                    gsl = pl.ds(j * 16, 16)
                    csl = pl.ds(D_FEAT + j * 16, 16)
                    ng = r1_v[slot, e, gsl] + r2_v[slot, e, gsl] + ep_v[slot, e, gsl]
                    nc_ = r1_v[slot, e, csl] + r2_v[slot, e, csl] + ep_v[slot, e, csl]
                    denom = (1.0 + jnp.exp(ng)) * (-1.0 - jnp.exp(nc_))
                    m_v[slot, e, gsl] = nc_ / denom

            dvec = dsti_v[pl.ds(i * _B, _B)]
            pltpu.async_copy(m_v.at[slot], agg_sh.at[dvec], ssems[slot],
                             add=True)

        def chunk_body(ci, carry):
            cbase = base_w + ci * _CHUNK
            pltpu.sync_copy(src_hbm.at[pl.ds(cbase, _CHUNK)], srci_v)
            pltpu.sync_copy(dst_hbm.at[pl.ds(cbase, _CHUNK)], dsti_v)
            issue(cbase, 0, 0)

            def pair_body(p, carry2):
                wok = jnp.logical_or(ci > 0, p >= 1)
                drain(0)
                issue(cbase, 2 * p + 1, 1)
                compute_scatter(2 * p, 0, wok)
                drain(1)

                @pl.when(p < _CB // 2 - 1)
                def _():
                    issue(cbase, 2 * p + 2, 0)

                compute_scatter(2 * p + 1, 1, wok)
                return carry2

            lax.fori_loop(0, _CB // 2, pair_body, 0)
            return carry

        lax.fori_loop(0, _NCHUNK, chunk_body, 0)

        tbase = base_w + _NCHUNK * _CHUNK
        pltpu.sync_copy(src_hbm.at[pl.ds(tbase, _B)], srci_v.at[pl.ds(0, _B)])
        pltpu.sync_copy(dst_hbm.at[pl.ds(tbase, _B)], dsti_v.at[pl.ds(0, _B)])
        issue(tbase, 0, 0)
        drain(0)
        compute_scatter(0, 0, jnp.bool_(True))

        wait_scatter(0)
        wait_scatter(1)
        plsc.subcore_barrier()
        pltpu.sync_copy(agg_sh.at[stripe], out_hbm.at[c, stripe])

    return k(xw1, xw2, eproj, src, dst, zeros)


def _post_body(x_ref, a0_ref, a1_ref, w_ref, b_ref, o_ref):
    agg = a0_ref[...] + a1_ref[...]
    o_ref[...] = (
        x_ref[...]
        + jnp.dot(agg, w_ref[...], preferred_element_type=jnp.float32)
        + b_ref[...]
    )


def _post(x, a0, a1, w_out, b_out):
    grid = (N_NODES // _XB,)
    return pl.pallas_call(
        _post_body,
        grid=grid,
        in_specs=[
            pl.BlockSpec((_XB, D_FEAT), lambda i: (i, 0)),
            pl.BlockSpec((_XB, D_FEAT), lambda i: (i, 0)),
            pl.BlockSpec((_XB, D_FEAT), lambda i: (i, 0)),
            pl.BlockSpec((D_FEAT, D_FEAT), lambda i: (0, 0)),
            pl.BlockSpec((1, D_FEAT), lambda i: (0, 0)),
        ],
        out_specs=pl.BlockSpec((_XB, D_FEAT), lambda i: (i, 0)),
        out_shape=jax.ShapeDtypeStruct((N_NODES, D_FEAT), jnp.float32),
    )(x, a0, a1, w_out, b_out)


def kernel(x, edge_index, edge_attr, W_msg, b_msg, W_out, b_out):
    x = x.astype(jnp.float32)
    src = edge_index[0].astype(jnp.int32)
    dst = edge_index[1].astype(jnp.int32)

    w1n = -W_msg[:D_FEAT]
    w2n = -W_msg[D_FEAT : 2 * D_FEAT]
    w3n = -W_msg[2 * D_FEAT :]

    xw1, xw2 = _mm2(x, w1n, w2n)
    ep = _eproj(edge_attr, w3n, (-b_msg).reshape(1, D_MSG))

    zeros = jnp.zeros((N_PAD, D_FEAT), jnp.float32)
    aggs = _sc_edge_kernel(xw1, xw2, ep, src, dst, zeros)

    return _post(
        x, aggs[0, :N_NODES], aggs[1, :N_NODES], W_out, b_out.reshape(1, D_FEAT)
    )
